# Initial kernel scaffold; baseline (speedup 1.0000x reference)
import jax
import jax.numpy as jnp
from jax.experimental import pallas as pl


def kernel(x, routing_weights, routing_indices, weights):
    raise NotImplementedError("write your pallas kernel here")


# speedup vs baseline: 16.7870x; 16.7870x over previous
import jax
import jax.numpy as jnp
from jax import lax
from jax.experimental import pallas as pl

TILE = 512


def _moe_body(ri_ref, rw_ref, w_ref, x_ref, y_ref):
    idx = ri_ref[0]
    w = w_ref[0]
    E = w.shape[0]

    eq = (idx[0][:, None] == lax.broadcasted_iota(jnp.int32, (idx.shape[1], E), 1))
    counts = jnp.sum(eq.astype(jnp.float32), axis=0, keepdims=True)
    maskf = (counts > 0.0).astype(jnp.float32)

    w_eff = jnp.dot(maskf, w, preferred_element_type=jnp.float32)

    t = lax.dot_general(w_eff, x_ref[0], (((1,), (1,)), ((), ())),
                        preferred_element_type=jnp.float32)

    y_ref[0] = rw_ref[0, 0][:, None] * t


def kernel(x, routing_weights, routing_indices, weights):
    B, S, D = x.shape
    E = weights.shape[0]
    rw = routing_weights.reshape(B, 1, S)
    ri = routing_indices.reshape(B, 1, S)
    wrows = jnp.transpose(weights[:, :B, :], (1, 0, 2))

    ntiles = S // TILE
    return pl.pallas_call(
        _moe_body,
        grid=(B, ntiles),
        in_specs=[
            pl.BlockSpec((1, 1, S), lambda j, t: (j, 0, 0)),
            pl.BlockSpec((1, 1, S), lambda j, t: (j, 0, 0)),
            pl.BlockSpec((1, E, D), lambda j, t: (j, 0, 0)),
            pl.BlockSpec((1, TILE, D), lambda j, t: (j, t, 0)),
        ],
        out_specs=pl.BlockSpec((1, S, TILE), lambda j, t: (j, 0, t)),
        out_shape=jax.ShapeDtypeStruct((B, S, S), x.dtype),
    )(ri, rw, wrows, x)


# speedup vs baseline: 17.0412x; 1.0151x over previous
import jax
import jax.numpy as jnp
from jax import lax
from jax.experimental import pallas as pl
from jax.experimental.pallas import tpu as pltpu

TILE = 512


def _moe_body(ri_ref, rw_ref, w_ref, x_ref, y_ref, weff_ref):

    @pl.when(pl.program_id(1) == 0)
    def _():
        idx = ri_ref[0]
        w = w_ref[0]
        E = w.shape[0]
        eq = (idx[0][:, None] == lax.broadcasted_iota(jnp.int32, (idx.shape[1], E), 1))
        counts = jnp.sum(eq.astype(jnp.float32), axis=0, keepdims=True)
        maskf = (counts > 0.0).astype(jnp.float32)
        weff_ref[...] = jnp.dot(maskf, w, preferred_element_type=jnp.float32)

    t = lax.dot_general(weff_ref[...], x_ref[0], (((1,), (1,)), ((), ())),
                        preferred_element_type=jnp.float32)

    y_ref[0] = rw_ref[0, 0][:, None] * t


def kernel(x, routing_weights, routing_indices, weights):
    B, S, D = x.shape
    E = weights.shape[0]
    rw = routing_weights.reshape(B, 1, S)
    ri = routing_indices.reshape(B, 1, S)
    wrows = jnp.transpose(weights[:, :B, :], (1, 0, 2))

    ntiles = S // TILE
    return pl.pallas_call(
        _moe_body,
        grid=(B, ntiles),
        in_specs=[
            pl.BlockSpec((1, 1, S), lambda j, t: (j, 0, 0)),
            pl.BlockSpec((1, 1, S), lambda j, t: (j, 0, 0)),
            pl.BlockSpec((1, E, D), lambda j, t: (j, 0, 0)),
            pl.BlockSpec((1, TILE, D), lambda j, t: (j, t, 0)),
        ],
        out_specs=pl.BlockSpec((1, S, TILE), lambda j, t: (j, 0, t)),
        out_shape=jax.ShapeDtypeStruct((B, S, S), x.dtype),
        scratch_shapes=[pltpu.VMEM((1, D), jnp.float32)],
    )(ri, rw, wrows, x)


# speedup vs baseline: 17.7704x; 1.0428x over previous
import jax
import jax.numpy as jnp
from jax import lax
from jax.experimental import pallas as pl
from jax.experimental.pallas import tpu as pltpu

TILE = 1024


def _moe_body(ri_ref, rw_ref, w_ref, x_ref, y_ref, weff_ref):

    @pl.when(pl.program_id(1) == 0)
    def _():
        idx = ri_ref[0]
        w = w_ref[0]
        E = w.shape[0]
        eq = (idx[0][:, None] == lax.broadcasted_iota(jnp.int32, (idx.shape[1], E), 1))
        counts = jnp.sum(eq.astype(jnp.float32), axis=0, keepdims=True)
        maskf = (counts > 0.0).astype(jnp.float32)
        weff_ref[...] = jnp.dot(maskf, w, preferred_element_type=jnp.float32)

    t = lax.dot_general(weff_ref[...], x_ref[0], (((1,), (1,)), ((), ())),
                        preferred_element_type=jnp.float32)

    y_ref[0] = rw_ref[0, 0][:, None] * t


def kernel(x, routing_weights, routing_indices, weights):
    B, S, D = x.shape
    E = weights.shape[0]
    rw = routing_weights.reshape(B, 1, S)
    ri = routing_indices.reshape(B, 1, S)
    wrows = jnp.transpose(weights[:, :B, :], (1, 0, 2))

    ntiles = S // TILE
    return pl.pallas_call(
        _moe_body,
        grid=(B, ntiles),
        in_specs=[
            pl.BlockSpec((1, 1, S), lambda j, t: (j, 0, 0)),
            pl.BlockSpec((1, 1, S), lambda j, t: (j, 0, 0)),
            pl.BlockSpec((1, E, D), lambda j, t: (j, 0, 0)),
            pl.BlockSpec((1, TILE, D), lambda j, t: (j, t, 0)),
        ],
        out_specs=pl.BlockSpec((1, S, TILE), lambda j, t: (j, 0, t)),
        out_shape=jax.ShapeDtypeStruct((B, S, S), x.dtype),
        scratch_shapes=[pltpu.VMEM((1, D), jnp.float32)],
    )(ri, rw, wrows, x)
